# Initial kernel scaffold; baseline (speedup 1.0000x reference)
import functools

import jax
import jax.numpy as jnp
from jax import lax
from jax.experimental import pallas as pl
from jax.experimental.pallas import tpu as pltpu
from jax.experimental.pallas import tpu_sc as plsc

D = 16
NC, NS, LANES = 2, 16, 16
NW = NC * NS
CHUNK = 2560
ROWS = CHUNK // 128


def _combo_body(bt, it, ut, wt, bias, c_out):
    code = lax.broadcasted_iota(jnp.int32, (8, 1), 0)
    f1 = (code & 1).astype(jnp.float32)
    f2 = ((code >> 1) & 1).astype(jnp.float32)
    f3 = ((code >> 2) & 1).astype(jnp.float32)
    pb = bt[0:1, :] + f1 * (bt[1:2, :] - bt[0:1, :])
    pi = it[0:1, :] + f2 * (it[1:2, :] - it[0:1, :])
    pu = ut[0:1, :] + f3 * (ut[1:2, :] - ut[0:1, :])
    comb = jnp.concatenate([pb, pi, pu], axis=1)
    c_out[...] = (
        jnp.dot(comb, wt[...], preferred_element_type=jnp.float32) + bias[...]
    )


def _combo_table(bt, it, ut, w_t, bias2d):
    return pl.pallas_call(
        _combo_body,
        out_shape=jax.ShapeDtypeStruct((8, D), jnp.float32),
    )(bt, it, ut, w_t, bias2d)


def _make_sc_lookup(n_tok):
    per_w = n_tok // NW
    n_chunk = per_w // CHUNK
    mesh = plsc.VectorSubcoreMesh(
        core_axis_name="c", subcore_axis_name="s", num_cores=NC, num_subcores=NS
    )

    @functools.partial(
        pl.kernel,
        mesh=mesh,
        out_type=jax.ShapeDtypeStruct((n_tok, D), jnp.float32),
        scratch_types=[
            pltpu.VMEM((CHUNK,), jnp.int32),
            pltpu.VMEM((CHUNK,), jnp.int32),
            pltpu.VMEM((CHUNK,), jnp.int32),
            pltpu.VMEM((ROWS, 128), jnp.int32),
            pltpu.VMEM((CHUNK, D), jnp.float32),
            pltpu.SemaphoreType.DMA,
        ],
    )
    def sc_lookup(f1_hbm, f2_hbm, f3_hbm, c_hbm, out_hbm,
                  f1_v, f2_v, f3_v, code_v, rows_v, sem):
        wid = lax.axis_index("s") * NC + lax.axis_index("c")
        base = wid * per_w

        def chunk_body(ci, carry):
            start = base + ci * CHUNK
            pltpu.sync_copy(f1_hbm.at[pl.ds(start, CHUNK)], f1_v)
            pltpu.sync_copy(f2_hbm.at[pl.ds(start, CHUNK)], f2_v)
            pltpu.sync_copy(f3_hbm.at[pl.ds(start, CHUNK)], f3_v)

            def row_body(j, carry2):
                for k in range(128 // LANES):
                    s = j * 128 + k * LANES
                    a = f1_v[pl.ds(s, LANES)]
                    bb = f2_v[pl.ds(s, LANES)]
                    cc = f3_v[pl.ds(s, LANES)]
                    code_v[j, pl.ds(k * LANES, LANES)] = a + bb * 2 + cc * 4
                return carry2

            lax.fori_loop(0, ROWS, row_body, 0)

            cps = []
            for j in range(ROWS):
                cps.append(
                    pltpu.async_copy(
                        c_hbm.at[code_v.at[j]],
                        rows_v.at[pl.ds(j * 128, 128)],
                        sem,
                    )
                )
            for cp in cps:
                cp.wait()
            pltpu.sync_copy(rows_v, out_hbm.at[pl.ds(start, CHUNK)])
            return carry

        lax.fori_loop(0, n_chunk, chunk_body, 0)

    return sc_lookup


def kernel(bold_flags, italic_flags, underline_flags,
           bold_table, italic_table, underline_table, W, b):
    B, T = bold_flags.shape
    n_tok = B * T
    combo = _combo_table(
        bold_table, italic_table, underline_table,
        W.T, b.reshape(1, D),
    )
    f1 = bold_flags.reshape(n_tok).astype(jnp.int32)
    f2 = italic_flags.reshape(n_tok).astype(jnp.int32)
    f3 = underline_flags.reshape(n_tok).astype(jnp.int32)
    out = _make_sc_lookup(n_tok)(f1, f2, f3, combo)
    return out.reshape(B, T, D)


# speedup vs baseline: 1.9004x; 1.9004x over previous
import functools

import jax
import jax.numpy as jnp
from jax import lax
from jax.experimental import pallas as pl
from jax.experimental.pallas import tpu as pltpu
from jax.experimental.pallas import tpu_sc as plsc

D = 16
NC, NS, LANES = 2, 16, 16
NW = NC * NS
CHUNK = 2560
ROWS = CHUNK // 128


def _combo_body(bt, it, ut, wt, bias, c_out):
    code = lax.broadcasted_iota(jnp.int32, (8, 1), 0)
    f1 = (code & 1).astype(jnp.float32)
    f2 = ((code >> 1) & 1).astype(jnp.float32)
    f3 = ((code >> 2) & 1).astype(jnp.float32)
    pb = bt[0:1, :] + f1 * (bt[1:2, :] - bt[0:1, :])
    pi = it[0:1, :] + f2 * (it[1:2, :] - it[0:1, :])
    pu = ut[0:1, :] + f3 * (ut[1:2, :] - ut[0:1, :])
    comb = jnp.concatenate([pb, pi, pu], axis=1)
    c_out[...] = (
        jnp.dot(comb, wt[...], preferred_element_type=jnp.float32) + bias[...]
    )


def _combo_table(bt, it, ut, w_t, bias2d):
    return pl.pallas_call(
        _combo_body,
        out_shape=jax.ShapeDtypeStruct((8, D), jnp.float32),
    )(bt, it, ut, w_t, bias2d)


def _make_sc_lookup(n_tok):
    per_w = n_tok // NW
    n_chunk = per_w // CHUNK
    mesh = plsc.VectorSubcoreMesh(
        core_axis_name="c", subcore_axis_name="s", num_cores=NC, num_subcores=NS
    )

    @functools.partial(
        pl.kernel,
        mesh=mesh,
        compiler_params=pltpu.CompilerParams(use_tc_tiling_on_sc=False),
        out_type=jax.ShapeDtypeStruct((n_tok, D), jnp.float32),
        scratch_types=[
            pltpu.VMEM((CHUNK,), jnp.int32),
            pltpu.VMEM((CHUNK,), jnp.int32),
            pltpu.VMEM((CHUNK,), jnp.int32),
            pltpu.VMEM((ROWS, 128), jnp.int32),
            pltpu.VMEM((CHUNK, D), jnp.float32),
            pltpu.SemaphoreType.DMA,
        ],
    )
    def sc_lookup(f1_hbm, f2_hbm, f3_hbm, c_hbm, out_hbm,
                  f1_v, f2_v, f3_v, code_v, rows_v, sem):
        wid = lax.axis_index("s") * NC + lax.axis_index("c")
        base = wid * per_w

        def chunk_body(ci, carry):
            start = base + ci * CHUNK
            pltpu.sync_copy(f1_hbm.at[pl.ds(start, CHUNK)], f1_v)
            pltpu.sync_copy(f2_hbm.at[pl.ds(start, CHUNK)], f2_v)
            pltpu.sync_copy(f3_hbm.at[pl.ds(start, CHUNK)], f3_v)

            def row_body(j, carry2):
                for k in range(128 // LANES):
                    s = j * 128 + k * LANES
                    a = f1_v[pl.ds(s, LANES)]
                    bb = f2_v[pl.ds(s, LANES)]
                    cc = f3_v[pl.ds(s, LANES)]
                    code_v[j, pl.ds(k * LANES, LANES)] = a + bb * 2 + cc * 4
                return carry2

            lax.fori_loop(0, ROWS, row_body, 0)

            cps = []
            for j in range(ROWS):
                cps.append(
                    pltpu.async_copy(
                        c_hbm.at[code_v.at[j]],
                        rows_v.at[pl.ds(j * 128, 128)],
                        sem,
                    )
                )
            for cp in cps:
                cp.wait()
            pltpu.sync_copy(rows_v, out_hbm.at[pl.ds(start, CHUNK)])
            return carry

        lax.fori_loop(0, n_chunk, chunk_body, 0)

    return sc_lookup


def kernel(bold_flags, italic_flags, underline_flags,
           bold_table, italic_table, underline_table, W, b):
    B, T = bold_flags.shape
    n_tok = B * T
    combo = _combo_table(
        bold_table, italic_table, underline_table,
        W.T, b.reshape(1, D),
    )
    f1 = bold_flags.reshape(n_tok).astype(jnp.int32)
    f2 = italic_flags.reshape(n_tok).astype(jnp.int32)
    f3 = underline_flags.reshape(n_tok).astype(jnp.int32)
    out = _make_sc_lookup(n_tok)(f1, f2, f3, combo)
    return out.reshape(B, T, D)


# speedup vs baseline: 12.9483x; 6.8134x over previous
import functools

import jax
import jax.numpy as jnp
from jax import lax
from jax.experimental import pallas as pl
from jax.experimental.pallas import tpu as pltpu
from jax.experimental.pallas import tpu_sc as plsc

D = 16
NC, NS, LANES = 2, 16, 16
NW = NC * NS
CHUNK = 2560
ROWS = CHUNK // 128


def _combo_body(bt, it, ut, wt, bias, c_out):
    code = lax.broadcasted_iota(jnp.int32, (8, 1), 0)
    f1 = (code & 1).astype(jnp.float32)
    f2 = ((code >> 1) & 1).astype(jnp.float32)
    f3 = ((code >> 2) & 1).astype(jnp.float32)
    pb = bt[0:1, :] + f1 * (bt[1:2, :] - bt[0:1, :])
    pi = it[0:1, :] + f2 * (it[1:2, :] - it[0:1, :])
    pu = ut[0:1, :] + f3 * (ut[1:2, :] - ut[0:1, :])
    comb = jnp.concatenate([pb, pi, pu], axis=1)
    c_out[...] = (
        jnp.dot(comb, wt[...], preferred_element_type=jnp.float32) + bias[...]
    )


def _combo_table(bt, it, ut, w_t, bias2d):
    return pl.pallas_call(
        _combo_body,
        out_shape=jax.ShapeDtypeStruct((8, D), jnp.float32),
    )(bt, it, ut, w_t, bias2d)


def _make_sc_lookup(n_tok):
    per_w = n_tok // NW
    n_chunk = per_w // CHUNK
    mesh = plsc.VectorSubcoreMesh(
        core_axis_name="c", subcore_axis_name="s", num_cores=NC, num_subcores=NS
    )

    @functools.partial(
        pl.kernel,
        mesh=mesh,
        compiler_params=pltpu.CompilerParams(use_tc_tiling_on_sc=False),
        out_type=jax.ShapeDtypeStruct((n_tok, D), jnp.float32),
        scratch_types=[
            pltpu.VMEM((CHUNK,), jnp.int32),
            pltpu.VMEM((CHUNK,), jnp.int32),
            pltpu.VMEM((CHUNK,), jnp.int32),
            pltpu.VMEM((CHUNK, D), jnp.float32),
            pltpu.VMEM((8, D), jnp.float32),
            pltpu.SemaphoreType.DMA,
        ],
    )
    def sc_lookup(f1_hbm, f2_hbm, f3_hbm, c_hbm, out_hbm,
                  f1_v, f2_v, f3_v, rows_v, c_v, sem):
        wid = lax.axis_index("s") * NC + lax.axis_index("c")
        base = wid * per_w
        pltpu.sync_copy(c_hbm, c_v)

        def chunk_body(ci, carry):
            start = base + ci * CHUNK
            pltpu.sync_copy(f1_hbm.at[pl.ds(start, CHUNK)], f1_v)
            pltpu.sync_copy(f2_hbm.at[pl.ds(start, CHUNK)], f2_v)
            pltpu.sync_copy(f3_hbm.at[pl.ds(start, CHUNK)], f3_v)

            def group_body(g, carry2):
                s = g * LANES
                a = f1_v[pl.ds(s, LANES)]
                bb = f2_v[pl.ds(s, LANES)]
                cc = f3_v[pl.ds(s, LANES)]
                code16 = a + bb * 2 + cc * 4
                for v in range(LANES):
                    rows_v[s + v, :] = c_v[code16[v], :]
                return carry2

            lax.fori_loop(0, CHUNK // LANES, group_body, 0)

            pltpu.sync_copy(rows_v, out_hbm.at[pl.ds(start, CHUNK)])
            return carry

        lax.fori_loop(0, n_chunk, chunk_body, 0)

    return sc_lookup


def kernel(bold_flags, italic_flags, underline_flags,
           bold_table, italic_table, underline_table, W, b):
    B, T = bold_flags.shape
    n_tok = B * T
    combo = _combo_table(
        bold_table, italic_table, underline_table,
        W.T, b.reshape(1, D),
    )
    f1 = bold_flags.reshape(n_tok).astype(jnp.int32)
    f2 = italic_flags.reshape(n_tok).astype(jnp.int32)
    f3 = underline_flags.reshape(n_tok).astype(jnp.int32)
    out = _make_sc_lookup(n_tok)(f1, f2, f3, combo)
    return out.reshape(B, T, D)


# speedup vs baseline: 12.9732x; 1.0019x over previous
import functools

import jax
import jax.numpy as jnp
from jax import lax
from jax.experimental import pallas as pl
from jax.experimental.pallas import tpu as pltpu
from jax.experimental.pallas import tpu_sc as plsc

D = 16
NC, NS, LANES = 2, 16, 16
NW = NC * NS
CHUNK = 2560
ROWS = CHUNK // 128


def _combo_body(bt, it, ut, wt, bias, c_out):
    code = lax.broadcasted_iota(jnp.int32, (8, 1), 0)
    f1 = (code & 1).astype(jnp.float32)
    f2 = ((code >> 1) & 1).astype(jnp.float32)
    f3 = ((code >> 2) & 1).astype(jnp.float32)
    pb = bt[0:1, :] + f1 * (bt[1:2, :] - bt[0:1, :])
    pi = it[0:1, :] + f2 * (it[1:2, :] - it[0:1, :])
    pu = ut[0:1, :] + f3 * (ut[1:2, :] - ut[0:1, :])
    comb = jnp.concatenate([pb, pi, pu], axis=1)
    c_out[...] = (
        jnp.dot(comb, wt[...], preferred_element_type=jnp.float32) + bias[...]
    )


def _combo_table(bt, it, ut, w_t, bias2d):
    return pl.pallas_call(
        _combo_body,
        out_shape=jax.ShapeDtypeStruct((8, D), jnp.float32),
    )(bt, it, ut, w_t, bias2d)


def _make_sc_lookup(n_tok):
    per_w = n_tok // NW
    n_chunk = per_w // CHUNK
    mesh = plsc.VectorSubcoreMesh(
        core_axis_name="c", subcore_axis_name="s", num_cores=NC, num_subcores=NS
    )

    @functools.partial(
        pl.kernel,
        mesh=mesh,
        compiler_params=pltpu.CompilerParams(use_tc_tiling_on_sc=False),
        out_type=jax.ShapeDtypeStruct((n_tok * D,), jnp.float32),
        scratch_types=[
            pltpu.VMEM((CHUNK,), jnp.int32),
            pltpu.VMEM((CHUNK,), jnp.int32),
            pltpu.VMEM((CHUNK,), jnp.int32),
            pltpu.VMEM((CHUNK * D,), jnp.float32),
            pltpu.VMEM((8 * D,), jnp.float32),
            pltpu.SemaphoreType.DMA,
        ],
    )
    def sc_lookup(f1_hbm, f2_hbm, f3_hbm, c_hbm, out_hbm,
                  f1_v, f2_v, f3_v, rows_v, c_v, sem):
        wid = lax.axis_index("s") * NC + lax.axis_index("c")
        base = wid * per_w
        pltpu.sync_copy(c_hbm, c_v)

        def chunk_body(ci, carry):
            start = base + ci * CHUNK
            pltpu.sync_copy(f1_hbm.at[pl.ds(start, CHUNK)], f1_v)
            pltpu.sync_copy(f2_hbm.at[pl.ds(start, CHUNK)], f2_v)
            pltpu.sync_copy(f3_hbm.at[pl.ds(start, CHUNK)], f3_v)

            def group_body(g, carry2):
                s = g * LANES
                a = f1_v[pl.ds(s, LANES)]
                bb = f2_v[pl.ds(s, LANES)]
                cc = f3_v[pl.ds(s, LANES)]
                code16 = a + bb * 2 + cc * 4
                for v in range(LANES):
                    rows_v[pl.ds((s + v) * D, D)] = c_v[pl.ds(code16[v] * D, D)]
                return carry2

            lax.fori_loop(0, CHUNK // LANES, group_body, 0)

            pltpu.sync_copy(rows_v, out_hbm.at[pl.ds(start * D, CHUNK * D)])
            return carry

        lax.fori_loop(0, n_chunk, chunk_body, 0)

    return sc_lookup


def kernel(bold_flags, italic_flags, underline_flags,
           bold_table, italic_table, underline_table, W, b):
    B, T = bold_flags.shape
    n_tok = B * T
    combo = _combo_table(
        bold_table, italic_table, underline_table,
        W.T, b.reshape(1, D),
    )
    f1 = bold_flags.reshape(n_tok).astype(jnp.int32)
    f2 = italic_flags.reshape(n_tok).astype(jnp.int32)
    f3 = underline_flags.reshape(n_tok).astype(jnp.int32)
    out = _make_sc_lookup(n_tok)(f1, f2, f3, combo.reshape(8 * D))
    return out.reshape(B, T, D)
